# Initial kernel scaffold; baseline (speedup 1.0000x reference)
import jax
import jax.numpy as jnp
from jax.experimental import pallas as pl


def kernel(x, edge_index, W1s, b1s, W1d, b1d, W2s, b2s, W2d, b2d):
    raise NotImplementedError("write your pallas kernel here")


# speedup vs baseline: 13.4584x; 13.4584x over previous
import functools

import jax
import jax.numpy as jnp
from jax import lax
from jax.experimental import pallas as pl
from jax.experimental.pallas import tpu as pltpu
from jax.experimental.pallas import tpu_sc as plsc

N = 10000
E = 320000
D_IN = 128
D_HID = 256
D_OUT = 128
ALPHA = 0.5

NPAD = 10240
NC = 2
NS = 16
K = 80
GROUP = 10
EPT = E // NS
ROWS_PT = NPAD // NS

_mesh = plsc.VectorSubcoreMesh(core_axis_name="c", subcore_axis_name="s")
_sc_params = pltpu.CompilerParams(needs_layout_passes=False,
                                  use_tc_tiling_on_sc=False)


def _zero_vmem_2d(ref, nrows, ncols):
    z16 = jnp.zeros((16,), jnp.float32)

    def body(i, _):
        def inner(j, __):
            ref[i, pl.ds(j * 16, 16)] = z16
            return __
        return lax.fori_loop(0, ncols // 16, inner, _)

    lax.fori_loop(0, nrows, body, None)


def _deg_body(de_ref, out_ref, hist, buf, tmp, accv, spm):
    c = lax.axis_index("c")
    s = lax.axis_index("s")
    z16 = jnp.zeros((16,), jnp.float32)
    ones16 = jnp.ones((16,), jnp.float32)

    def zh(i, _):
        hist[pl.ds(i * 16, 16)] = z16
        return _
    lax.fori_loop(0, NPAD // 16, zh, None)

    base = c * E + s * EPT

    def stage(r, _):
        pltpu.sync_copy(de_ref.at[pl.ds(base + r * 800, 800)], buf)

        def upd(j, __):
            idx = buf[pl.ds(j * 16, 16)]
            plsc.addupdate_scatter(hist, [idx], ones16)
            return __
        return lax.fori_loop(0, 50, upd, _)
    lax.fori_loop(0, EPT // 800, stage, None)

    pltpu.sync_copy(hist, spm.at[s])
    plsc.subcore_barrier()

    def za(i, _):
        accv[pl.ds(i * 16, 16)] = z16
        return _
    lax.fori_loop(0, ROWS_PT // 16, za, None)

    for p in range(NS):
        pltpu.sync_copy(spm.at[p, pl.ds(s * ROWS_PT, ROWS_PT)], tmp)

        def acc_add(k, _):
            sl = pl.ds(k * 16, 16)
            accv[sl] = accv[sl] + tmp[sl]
            return _
        lax.fori_loop(0, ROWS_PT // 16, acc_add, None)

    pltpu.sync_copy(accv, out_ref.at[c, pl.ds(s * ROWS_PT, ROWS_PT)])


_deg_kernel = functools.partial(
    pl.kernel,
    out_type=jax.ShapeDtypeStruct((NC, NPAD), jnp.float32),
    mesh=_mesh,
    scratch_types=[
        pltpu.VMEM((NPAD,), jnp.float32),
        pltpu.VMEM((800,), jnp.int32),
        pltpu.VMEM((ROWS_PT,), jnp.float32),
        pltpu.VMEM((ROWS_PT,), jnp.float32),
        pltpu.VMEM_SHARED((NS, NPAD), jnp.float32),
    ],
    compiler_params=_sc_params,
)(_deg_body)


def _agg_body(srcs_ref, ge_ref, se_ref, out_ref, acc, bufg, bufs, rows, zbuf,
              sem):
    c = lax.axis_index("c")
    s = lax.axis_index("s")

    _zero_vmem_2d(zbuf, 128, 128)
    for k in range(ROWS_PT // 128):
        pltpu.sync_copy(zbuf, acc.at[pl.ds(s * ROWS_PT + k * 128, 128)])
    plsc.subcore_barrier()

    base = c * (E // K) + s * (EPT // K)

    def stage(r, _):
        pltpu.sync_copy(ge_ref.at[pl.ds(base + r * GROUP, GROUP)], bufg)
        pltpu.sync_copy(se_ref.at[pl.ds(base + r * GROUP, GROUP)], bufs)

        def chunk(j, __):
            pltpu.async_copy(srcs_ref.at[bufg.at[j]], rows, sem).wait()
            pltpu.sync_copy(rows, acc.at[bufs.at[j]], add=True)
            return __
        return lax.fori_loop(0, GROUP, chunk, _)
    lax.fori_loop(0, EPT // K // GROUP, stage, None)

    plsc.subcore_barrier()
    for k in range(ROWS_PT // 128):
        sl = pl.ds(s * ROWS_PT + k * 128, 128)
        pltpu.sync_copy(acc.at[sl], out_ref.at[c].at[sl])


_agg_kernel = functools.partial(
    pl.kernel,
    out_type=jax.ShapeDtypeStruct((NC, NPAD, D_IN), jnp.float32),
    mesh=_mesh,
    scratch_types=[
        pltpu.VMEM_SHARED((NPAD, D_IN), jnp.float32),
        pltpu.VMEM((GROUP, K), jnp.int32),
        pltpu.VMEM((GROUP, K), jnp.int32),
        pltpu.VMEM((K, D_IN), jnp.float32),
        pltpu.VMEM((128, 128), jnp.float32),
        pltpu.SemaphoreType.DMA,
    ],
    compiler_params=_sc_params,
)(_agg_body)


def _inv(d):
    return jnp.where(d > 0.0, lax.rsqrt(d), 0.0)


def _pre_body(x_ref, od_ref, id_ref, out_ref):
    x = x_ref[...]
    out_ref[0] = _inv(id_ref[...]) * x
    out_ref[1] = _inv(od_ref[...]) * x


def _mid_body(agg_ref, od_ref, id_ref, w1s_ref, b1s_ref, w1d_ref, b1d_ref,
              w2s_ref, w2d_ref, out_ref):
    oi = _inv(od_ref[...])
    ii = _inv(id_ref[...])
    a0 = oi * agg_ref[0]
    a1 = ii * agg_ref[1]
    h = ALPHA * (jnp.dot(a0, w1s_ref[...], preferred_element_type=jnp.float32)
                 + b1s_ref[...])
    h += (1.0 - ALPHA) * (jnp.dot(a1, w1d_ref[...],
                                  preferred_element_type=jnp.float32)
                          + b1d_ref[...])
    h = jnp.maximum(h, 0.0)
    out_ref[0] = ii * jnp.dot(h, w2s_ref[...],
                              preferred_element_type=jnp.float32)
    out_ref[1] = oi * jnp.dot(h, w2d_ref[...],
                              preferred_element_type=jnp.float32)


def _fin_body(agg_ref, od_ref, id_ref, b2s_ref, b2d_ref, out_ref):
    oi = _inv(od_ref[...])
    ii = _inv(id_ref[...])
    out_ref[...] = (ALPHA * (oi * agg_ref[0] + b2s_ref[...])
                    + (1.0 - ALPHA) * (ii * agg_ref[1] + b2d_ref[...]))


_BN = 1000


def _col(i):
    return (i, 0)


def _col3(i):
    return (0, i, 0)


def _rep2(i):
    return (0, 0)


@jax.jit
def kernel(x, edge_index, W1s, b1s, W1d, b1d, W2s, b2s, W2d, b2d):
    row = edge_index[0].astype(jnp.int32)
    col = edge_index[1].astype(jnp.int32)

    ge = jnp.concatenate([col, row + N]).reshape(2 * E // K, K)
    se = jnp.concatenate([row, col]).reshape(2 * E // K, K)
    de = jnp.concatenate([row, col])

    degs = _deg_kernel(de)
    od = degs[0, :N].reshape(N, 1)
    idg = degs[1, :N].reshape(N, 1)

    grid = N // _BN
    deg_spec = pl.BlockSpec((_BN, 1), _col)
    agg_spec = pl.BlockSpec((NC, _BN, D_IN), _col3)

    srcs1 = pl.pallas_call(
        _pre_body,
        grid=(grid,),
        in_specs=[pl.BlockSpec((_BN, D_IN), _col), deg_spec, deg_spec],
        out_specs=pl.BlockSpec((NC, _BN, D_IN), _col3),
        out_shape=jax.ShapeDtypeStruct((NC, N, D_IN), jnp.float32),
    )(x, od, idg)

    agg1 = _agg_kernel(srcs1.reshape(NC * N, D_IN), ge, se)

    srcs2 = pl.pallas_call(
        _mid_body,
        grid=(grid,),
        in_specs=[
            agg_spec, deg_spec, deg_spec,
            pl.BlockSpec((D_IN, D_HID), _rep2),
            pl.BlockSpec((1, D_HID), _rep2),
            pl.BlockSpec((D_IN, D_HID), _rep2),
            pl.BlockSpec((1, D_HID), _rep2),
            pl.BlockSpec((D_HID, D_OUT), _rep2),
            pl.BlockSpec((D_HID, D_OUT), _rep2),
        ],
        out_specs=pl.BlockSpec((NC, _BN, D_OUT), _col3),
        out_shape=jax.ShapeDtypeStruct((NC, N, D_OUT), jnp.float32),
    )(agg1, od, idg, W1s.T, b1s.reshape(1, -1), W1d.T, b1d.reshape(1, -1),
      W2s.T, W2d.T)

    agg2 = _agg_kernel(srcs2.reshape(NC * N, D_OUT), ge, se)

    out = pl.pallas_call(
        _fin_body,
        grid=(grid,),
        in_specs=[
            agg_spec, deg_spec, deg_spec,
            pl.BlockSpec((1, D_OUT), _rep2),
            pl.BlockSpec((1, D_OUT), _rep2),
        ],
        out_specs=pl.BlockSpec((_BN, D_OUT), _col),
        out_shape=jax.ShapeDtypeStruct((N, D_OUT), jnp.float32),
    )(agg2, od, idg, b2s.reshape(1, -1), b2d.reshape(1, -1))

    return out


# speedup vs baseline: 18.0204x; 1.3390x over previous
import functools

import jax
import jax.numpy as jnp
from jax import lax
from jax.experimental import pallas as pl
from jax.experimental.pallas import tpu as pltpu
from jax.experimental.pallas import tpu_sc as plsc

N = 10000
E = 320000
D_IN = 128
D_HID = 256
D_OUT = 128
ALPHA = 0.5

NPAD = 10240
NC = 2
NS = 16
K = 80
GROUP = 10
EPT = E // NS
ROWS_PT = NPAD // NS

_mesh = plsc.VectorSubcoreMesh(core_axis_name="c", subcore_axis_name="s")
_sc_params = pltpu.CompilerParams(needs_layout_passes=False,
                                  use_tc_tiling_on_sc=False)


def _zero_vmem_2d(ref, nrows, ncols):
    z16 = jnp.zeros((16,), jnp.float32)

    def body(i, _):
        def inner(j, __):
            ref[i, pl.ds(j * 16, 16)] = z16
            return __
        return lax.fori_loop(0, ncols // 16, inner, _)

    lax.fori_loop(0, nrows, body, None)


def _deg_body(de_ref, out_ref, hist, buf, tmp, accv, spm):
    c = lax.axis_index("c")
    s = lax.axis_index("s")
    z16 = jnp.zeros((16,), jnp.float32)
    ones16 = jnp.ones((16,), jnp.float32)

    def zh(i, _):
        hist[pl.ds(i * 16, 16)] = z16
        return _
    lax.fori_loop(0, NPAD // 16, zh, None)

    base = c * E + s * EPT

    def stage(r, _):
        pltpu.sync_copy(de_ref.at[pl.ds(base + r * 800, 800)], buf)

        def upd(j, __):
            idx = buf[pl.ds(j * 16, 16)]
            plsc.addupdate_scatter(hist, [idx], ones16)
            return __
        return lax.fori_loop(0, 50, upd, _)
    lax.fori_loop(0, EPT // 800, stage, None)

    pltpu.sync_copy(hist, spm.at[s])
    plsc.subcore_barrier()

    def za(i, _):
        accv[pl.ds(i * 16, 16)] = z16
        return _
    lax.fori_loop(0, ROWS_PT // 16, za, None)

    for p in range(NS):
        pltpu.sync_copy(spm.at[p, pl.ds(s * ROWS_PT, ROWS_PT)], tmp)

        def acc_add(k, _):
            sl = pl.ds(k * 16, 16)
            accv[sl] = accv[sl] + tmp[sl]
            return _
        lax.fori_loop(0, ROWS_PT // 16, acc_add, None)

    pltpu.sync_copy(accv, out_ref.at[c, pl.ds(s * ROWS_PT, ROWS_PT)])


_deg_kernel = functools.partial(
    pl.kernel,
    out_type=jax.ShapeDtypeStruct((NC, NPAD), jnp.float32),
    mesh=_mesh,
    scratch_types=[
        pltpu.VMEM((NPAD,), jnp.float32),
        pltpu.VMEM((800,), jnp.int32),
        pltpu.VMEM((ROWS_PT,), jnp.float32),
        pltpu.VMEM((ROWS_PT,), jnp.float32),
        pltpu.VMEM_SHARED((NS, NPAD), jnp.float32),
    ],
    compiler_params=_sc_params,
)(_deg_body)


G = 50


def _agg_body(srcs_ref, ge_ref, se_ref, out_ref, acc, bufg, bufs, rows_a,
              rows_b, zbuf, semg):
    c = lax.axis_index("c")
    s = lax.axis_index("s")
    n_chunks = EPT // K
    base = (c * NS + s) * n_chunks

    _zero_vmem_2d(zbuf, 64, 128)
    for k in range(ROWS_PT // 64):
        pltpu.sync_copy(zbuf, acc.at[pl.ds(s * ROWS_PT + k * 64, 64)])
    plsc.subcore_barrier()

    def gstart(t, dst):
        pltpu.async_copy(srcs_ref.at[bufg.at[t]], dst, semg)

    def gwait(t, dst):
        pltpu.make_async_copy(srcs_ref.at[bufg.at[t]], dst, semg).wait()

    def rnd(r, _):
        pltpu.sync_copy(ge_ref.at[pl.ds(base + r * G, G)], bufg)
        pltpu.sync_copy(se_ref.at[pl.ds(base + r * G, G)], bufs)

        gstart(0, rows_a)

        def pair(i, __):
            t = 2 * i
            gwait(t, rows_a)
            gstart(t + 1, rows_b)
            pltpu.sync_copy(rows_a, acc.at[bufs.at[t]], add=True)
            gwait(t + 1, rows_b)

            @pl.when(i < G // 2 - 1)
            def _():
                gstart(t + 2, rows_a)

            pltpu.sync_copy(rows_b, acc.at[bufs.at[t + 1]], add=True)
            return __
        return lax.fori_loop(0, G // 2, pair, _)
    lax.fori_loop(0, n_chunks // G, rnd, None)

    plsc.subcore_barrier()
    for k in range(ROWS_PT // 128):
        sl = pl.ds(s * ROWS_PT + k * 128, 128)
        pltpu.sync_copy(acc.at[sl], out_ref.at[c].at[sl])


_agg_kernel = functools.partial(
    pl.kernel,
    out_type=jax.ShapeDtypeStruct((NC, NPAD, D_IN), jnp.float32),
    mesh=_mesh,
    scratch_types=[
        pltpu.VMEM_SHARED((NPAD, D_IN), jnp.float32),
        pltpu.VMEM((G, K), jnp.int32),
        pltpu.VMEM((G, K), jnp.int32),
        pltpu.VMEM((K, D_IN), jnp.float32),
        pltpu.VMEM((K, D_IN), jnp.float32),
        pltpu.VMEM((64, 128), jnp.float32),
        pltpu.SemaphoreType.DMA,
    ],
    compiler_params=_sc_params,
)(_agg_body)


def _inv(d):
    return jnp.where(d > 0.0, lax.rsqrt(d), 0.0)


def _pre_body(x_ref, od_ref, id_ref, out_ref):
    x = x_ref[...]
    out_ref[0] = _inv(id_ref[...]) * x
    out_ref[1] = _inv(od_ref[...]) * x


def _mid_body(agg_ref, od_ref, id_ref, w1s_ref, b1s_ref, w1d_ref, b1d_ref,
              w2s_ref, w2d_ref, out_ref):
    oi = _inv(od_ref[...])
    ii = _inv(id_ref[...])
    a0 = oi * agg_ref[0]
    a1 = ii * agg_ref[1]
    h = ALPHA * (jnp.dot(a0, w1s_ref[...], preferred_element_type=jnp.float32)
                 + b1s_ref[...])
    h += (1.0 - ALPHA) * (jnp.dot(a1, w1d_ref[...],
                                  preferred_element_type=jnp.float32)
                          + b1d_ref[...])
    h = jnp.maximum(h, 0.0)
    out_ref[0] = ii * jnp.dot(h, w2s_ref[...],
                              preferred_element_type=jnp.float32)
    out_ref[1] = oi * jnp.dot(h, w2d_ref[...],
                              preferred_element_type=jnp.float32)


def _fin_body(agg_ref, od_ref, id_ref, b2s_ref, b2d_ref, out_ref):
    oi = _inv(od_ref[...])
    ii = _inv(id_ref[...])
    out_ref[...] = (ALPHA * (oi * agg_ref[0] + b2s_ref[...])
                    + (1.0 - ALPHA) * (ii * agg_ref[1] + b2d_ref[...]))


_BN = 1000


def _col(i):
    return (i, 0)


def _col3(i):
    return (0, i, 0)


def _rep2(i):
    return (0, 0)


@jax.jit
def kernel(x, edge_index, W1s, b1s, W1d, b1d, W2s, b2s, W2d, b2d):
    row = edge_index[0].astype(jnp.int32)
    col = edge_index[1].astype(jnp.int32)

    ge = jnp.concatenate([col, row + N]).reshape(2 * E // K, K)
    se = jnp.concatenate([row, col]).reshape(2 * E // K, K)
    de = jnp.concatenate([row, col])

    degs = _deg_kernel(de)
    od = degs[0, :N].reshape(N, 1)
    idg = degs[1, :N].reshape(N, 1)

    grid = N // _BN
    deg_spec = pl.BlockSpec((_BN, 1), _col)
    agg_spec = pl.BlockSpec((NC, _BN, D_IN), _col3)

    srcs1 = pl.pallas_call(
        _pre_body,
        grid=(grid,),
        in_specs=[pl.BlockSpec((_BN, D_IN), _col), deg_spec, deg_spec],
        out_specs=pl.BlockSpec((NC, _BN, D_IN), _col3),
        out_shape=jax.ShapeDtypeStruct((NC, N, D_IN), jnp.float32),
    )(x, od, idg)

    agg1 = _agg_kernel(srcs1.reshape(NC * N, D_IN), ge, se)

    srcs2 = pl.pallas_call(
        _mid_body,
        grid=(grid,),
        in_specs=[
            agg_spec, deg_spec, deg_spec,
            pl.BlockSpec((D_IN, D_HID), _rep2),
            pl.BlockSpec((1, D_HID), _rep2),
            pl.BlockSpec((D_IN, D_HID), _rep2),
            pl.BlockSpec((1, D_HID), _rep2),
            pl.BlockSpec((D_HID, D_OUT), _rep2),
            pl.BlockSpec((D_HID, D_OUT), _rep2),
        ],
        out_specs=pl.BlockSpec((NC, _BN, D_OUT), _col3),
        out_shape=jax.ShapeDtypeStruct((NC, N, D_OUT), jnp.float32),
    )(agg1, od, idg, W1s.T, b1s.reshape(1, -1), W1d.T, b1d.reshape(1, -1),
      W2s.T, W2d.T)

    agg2 = _agg_kernel(srcs2.reshape(NC * N, D_OUT), ge, se)

    out = pl.pallas_call(
        _fin_body,
        grid=(grid,),
        in_specs=[
            agg_spec, deg_spec, deg_spec,
            pl.BlockSpec((1, D_OUT), _rep2),
            pl.BlockSpec((1, D_OUT), _rep2),
        ],
        out_specs=pl.BlockSpec((_BN, D_OUT), _col),
        out_shape=jax.ShapeDtypeStruct((N, D_OUT), jnp.float32),
    )(agg2, od, idg, b2s.reshape(1, -1), b2d.reshape(1, -1))

    return out


# speedup vs baseline: 25.2693x; 1.4023x over previous
import functools

import jax
import jax.numpy as jnp
from jax import lax
from jax.experimental import pallas as pl
from jax.experimental.pallas import tpu as pltpu
from jax.experimental.pallas import tpu_sc as plsc

N = 10000
E = 320000
D_IN = 128
D_HID = 256
D_OUT = 128
ALPHA = 0.5

NPAD = 10240
NC = 2
NS = 16
K = 80
GROUP = 10
EPT = E // NS
ROWS_PT = NPAD // NS

_mesh = plsc.VectorSubcoreMesh(core_axis_name="c", subcore_axis_name="s")
_sc_params = pltpu.CompilerParams(needs_layout_passes=False,
                                  use_tc_tiling_on_sc=False)


def _zero_vmem_2d(ref, nrows, ncols):
    z16 = jnp.zeros((16,), jnp.float32)

    def body(i, _):
        def inner(j, __):
            ref[i, pl.ds(j * 16, 16)] = z16
            return __
        return lax.fori_loop(0, ncols // 16, inner, _)

    lax.fori_loop(0, nrows, body, None)


def _deg_body(de_ref, out_ref, hist, buf, tmp, accv, spm):
    c = lax.axis_index("c")
    s = lax.axis_index("s")
    z16 = jnp.zeros((16,), jnp.float32)
    ones16 = jnp.ones((16,), jnp.float32)

    def zh(i, _):
        hist[pl.ds(i * 16, 16)] = z16
        return _
    lax.fori_loop(0, NPAD // 16, zh, None)

    base = c * E + s * EPT

    def stage(r, _):
        pltpu.sync_copy(de_ref.at[pl.ds(base + r * 800, 800)], buf)

        def upd(j, __):
            idx = buf[pl.ds(j * 16, 16)]
            plsc.addupdate_scatter(hist, [idx], ones16)
            return __
        return lax.fori_loop(0, 50, upd, _)
    lax.fori_loop(0, EPT // 800, stage, None)

    pltpu.sync_copy(hist, spm.at[s])
    plsc.subcore_barrier()

    def za(i, _):
        accv[pl.ds(i * 16, 16)] = z16
        return _
    lax.fori_loop(0, ROWS_PT // 16, za, None)

    for p in range(NS):
        pltpu.sync_copy(spm.at[p, pl.ds(s * ROWS_PT, ROWS_PT)], tmp)

        def acc_add(k, _):
            sl = pl.ds(k * 16, 16)
            accv[sl] = accv[sl] + tmp[sl]
            return _
        lax.fori_loop(0, ROWS_PT // 16, acc_add, None)

    pltpu.sync_copy(accv, out_ref.at[c, pl.ds(s * ROWS_PT, ROWS_PT)])


_deg_kernel = functools.partial(
    pl.kernel,
    out_type=jax.ShapeDtypeStruct((NC, NPAD), jnp.float32),
    mesh=_mesh,
    scratch_types=[
        pltpu.VMEM((NPAD,), jnp.float32),
        pltpu.VMEM((800,), jnp.int32),
        pltpu.VMEM((ROWS_PT,), jnp.float32),
        pltpu.VMEM((ROWS_PT,), jnp.float32),
        pltpu.VMEM_SHARED((NS, NPAD), jnp.float32),
    ],
    compiler_params=_sc_params,
)(_deg_body)


G = 25
NBUF = 3


def _agg_body(srcs_ref, ge_ref, se_ref, out_ref, acc, bufg, bufs, rows, zbuf,
              semg, sems):
    c = lax.axis_index("c")
    s = lax.axis_index("s")
    n_chunks = EPT // K
    base = (c * NS + s) * n_chunks

    _zero_vmem_2d(zbuf, 16, 128)
    for k in range(ROWS_PT // 16):
        pltpu.sync_copy(zbuf, acc.at[pl.ds(s * ROWS_PT + k * 16, 16)])
    plsc.subcore_barrier()

    def rbuf(t):
        return rows.at[lax.rem(t, NBUF)]

    def gstart(t, dst):
        pltpu.async_copy(srcs_ref.at[bufg.at[t]], dst, semg)

    def gwait(t, dst):
        pltpu.make_async_copy(srcs_ref.at[bufg.at[t]], dst, semg).wait()

    def sstart(t):
        pltpu.async_copy(rbuf(t), acc.at[bufs.at[t]], sems, add=True)

    def swait(t):
        pltpu.make_async_copy(rbuf(t), acc.at[bufs.at[t]], sems).wait()

    def rnd(r, _):
        pltpu.sync_copy(ge_ref.at[pl.ds(base + r * G, G)], bufg)
        pltpu.sync_copy(se_ref.at[pl.ds(base + r * G, G)], bufs)
        gstart(0, rbuf(0))
        gstart(1, rbuf(1))

        def chunk(j, __):
            gwait(j, rbuf(j))
            sstart(j)

            @pl.when(j >= 1)
            def _():
                swait(j - 1)

            @pl.when(j <= G - 3)
            def _():
                gstart(j + 2, rbuf(j + 2))
            return __
        lax.fori_loop(0, G, chunk, None)
        swait(G - 1)
        return _
    lax.fori_loop(0, n_chunks // G, rnd, None)

    plsc.subcore_barrier()
    for k in range(ROWS_PT // 128):
        sl = pl.ds(s * ROWS_PT + k * 128, 128)
        pltpu.sync_copy(acc.at[sl], out_ref.at[c].at[sl])


_agg_kernel = functools.partial(
    pl.kernel,
    out_type=jax.ShapeDtypeStruct((NC, NPAD, D_IN), jnp.float32),
    mesh=_mesh,
    scratch_types=[
        pltpu.VMEM_SHARED((NPAD, D_IN), jnp.float32),
        pltpu.VMEM((G, K), jnp.int32),
        pltpu.VMEM((G, K), jnp.int32),
        pltpu.VMEM((NBUF, K, D_IN), jnp.float32),
        pltpu.VMEM((16, 128), jnp.float32),
        pltpu.SemaphoreType.DMA,
        pltpu.SemaphoreType.DMA,
    ],
    compiler_params=_sc_params,
)(_agg_body)


def _inv(d):
    return jnp.where(d > 0.0, lax.rsqrt(d), 0.0)


def _pre_body(x_ref, od_ref, id_ref, out_ref):
    x = x_ref[...]
    out_ref[0] = _inv(id_ref[...]) * x
    out_ref[1] = _inv(od_ref[...]) * x


def _mid_body(agg_ref, od_ref, id_ref, w1s_ref, b1s_ref, w1d_ref, b1d_ref,
              w2s_ref, w2d_ref, out_ref):
    oi = _inv(od_ref[...])
    ii = _inv(id_ref[...])
    a0 = oi * agg_ref[0]
    a1 = ii * agg_ref[1]
    h = ALPHA * (jnp.dot(a0, w1s_ref[...], preferred_element_type=jnp.float32)
                 + b1s_ref[...])
    h += (1.0 - ALPHA) * (jnp.dot(a1, w1d_ref[...],
                                  preferred_element_type=jnp.float32)
                          + b1d_ref[...])
    h = jnp.maximum(h, 0.0)
    out_ref[0] = ii * jnp.dot(h, w2s_ref[...],
                              preferred_element_type=jnp.float32)
    out_ref[1] = oi * jnp.dot(h, w2d_ref[...],
                              preferred_element_type=jnp.float32)


def _fin_body(agg_ref, od_ref, id_ref, b2s_ref, b2d_ref, out_ref):
    oi = _inv(od_ref[...])
    ii = _inv(id_ref[...])
    out_ref[...] = (ALPHA * (oi * agg_ref[0] + b2s_ref[...])
                    + (1.0 - ALPHA) * (ii * agg_ref[1] + b2d_ref[...]))


_BN = 1000


def _col(i):
    return (i, 0)


def _col3(i):
    return (0, i, 0)


def _rep2(i):
    return (0, 0)


@jax.jit
def kernel(x, edge_index, W1s, b1s, W1d, b1d, W2s, b2s, W2d, b2d):
    row = edge_index[0].astype(jnp.int32)
    col = edge_index[1].astype(jnp.int32)

    ge = jnp.concatenate([col, row + N]).reshape(2 * E // K, K)
    se = jnp.concatenate([row, col]).reshape(2 * E // K, K)
    de = jnp.concatenate([row, col])

    degs = _deg_kernel(de)
    od = degs[0, :N].reshape(N, 1)
    idg = degs[1, :N].reshape(N, 1)

    grid = N // _BN
    deg_spec = pl.BlockSpec((_BN, 1), _col)
    agg_spec = pl.BlockSpec((NC, _BN, D_IN), _col3)

    srcs1 = pl.pallas_call(
        _pre_body,
        grid=(grid,),
        in_specs=[pl.BlockSpec((_BN, D_IN), _col), deg_spec, deg_spec],
        out_specs=pl.BlockSpec((NC, _BN, D_IN), _col3),
        out_shape=jax.ShapeDtypeStruct((NC, N, D_IN), jnp.float32),
    )(x, od, idg)

    agg1 = _agg_kernel(srcs1.reshape(NC * N, D_IN), ge, se)

    srcs2 = pl.pallas_call(
        _mid_body,
        grid=(grid,),
        in_specs=[
            agg_spec, deg_spec, deg_spec,
            pl.BlockSpec((D_IN, D_HID), _rep2),
            pl.BlockSpec((1, D_HID), _rep2),
            pl.BlockSpec((D_IN, D_HID), _rep2),
            pl.BlockSpec((1, D_HID), _rep2),
            pl.BlockSpec((D_HID, D_OUT), _rep2),
            pl.BlockSpec((D_HID, D_OUT), _rep2),
        ],
        out_specs=pl.BlockSpec((NC, _BN, D_OUT), _col3),
        out_shape=jax.ShapeDtypeStruct((NC, N, D_OUT), jnp.float32),
    )(agg1, od, idg, W1s.T, b1s.reshape(1, -1), W1d.T, b1d.reshape(1, -1),
      W2s.T, W2d.T)

    agg2 = _agg_kernel(srcs2.reshape(NC * N, D_OUT), ge, se)

    out = pl.pallas_call(
        _fin_body,
        grid=(grid,),
        in_specs=[
            agg_spec, deg_spec, deg_spec,
            pl.BlockSpec((1, D_OUT), _rep2),
            pl.BlockSpec((1, D_OUT), _rep2),
        ],
        out_specs=pl.BlockSpec((_BN, D_OUT), _col),
        out_shape=jax.ShapeDtypeStruct((N, D_OUT), jnp.float32),
    )(agg2, od, idg, b2s.reshape(1, -1), b2d.reshape(1, -1))

    return out


# speedup vs baseline: 26.9983x; 1.0684x over previous
import functools

import jax
import jax.numpy as jnp
from jax import lax
from jax.experimental import pallas as pl
from jax.experimental.pallas import tpu as pltpu
from jax.experimental.pallas import tpu_sc as plsc

N = 10000
E = 320000
D_IN = 128
D_HID = 256
D_OUT = 128
ALPHA = 0.5

NPAD = 10240
NC = 2
NS = 16
K = 100
EPT = E // NS
ROWS_PT = NPAD // NS

_mesh = plsc.VectorSubcoreMesh(core_axis_name="c", subcore_axis_name="s")
_sc_params = pltpu.CompilerParams(needs_layout_passes=False,
                                  use_tc_tiling_on_sc=False)


def _zero_vmem_2d(ref, nrows, ncols):
    z16 = jnp.zeros((16,), jnp.float32)

    def body(i, _):
        def inner(j, __):
            ref[i, pl.ds(j * 16, 16)] = z16
            return __
        return lax.fori_loop(0, ncols // 16, inner, _)

    lax.fori_loop(0, nrows, body, None)


def _deg_body(de_ref, out_ref, hist, buf, tmp, accv, spm, semi, semt):
    c = lax.axis_index("c")
    s = lax.axis_index("s")
    z16 = jnp.zeros((16,), jnp.float32)
    ones16 = jnp.ones((16,), jnp.float32)

    def zh(i, _):
        hist[pl.ds(i * 16, 16)] = z16
        return _
    lax.fori_loop(0, NPAD // 16, zh, None)

    base = c * E + s * EPT
    n_stages = EPT // 800

    def istart(r):
        pltpu.async_copy(de_ref.at[pl.ds(base + r * 800, 800)],
                         buf.at[lax.rem(r, 2)], semi)

    def iwait(r):
        pltpu.make_async_copy(de_ref.at[pl.ds(base + r * 800, 800)],
                              buf.at[lax.rem(r, 2)], semi).wait()

    istart(0)

    def stage(r, _):
        iwait(r)

        @pl.when(r < n_stages - 1)
        def _():
            istart(r + 1)

        bslot = buf.at[lax.rem(r, 2)]

        def upd(j, __):
            idx = bslot[pl.ds(j * 16, 16)]
            plsc.addupdate_scatter(hist, [idx], ones16)
            return __
        return lax.fori_loop(0, 50, upd, _)
    lax.fori_loop(0, n_stages, stage, None)

    pltpu.sync_copy(hist, spm.at[s])
    plsc.subcore_barrier()

    for p in range(NS):
        pltpu.async_copy(spm.at[p, pl.ds(s * ROWS_PT, ROWS_PT)], tmp.at[p],
                         semt)
    for p in range(NS):
        pltpu.make_async_copy(spm.at[p, pl.ds(s * ROWS_PT, ROWS_PT)],
                              tmp.at[p], semt).wait()

    def acc_add(k, _):
        sl = pl.ds(k * 16, 16)
        v = tmp[0, sl]
        for p in range(1, NS):
            v = v + tmp[p, sl]
        accv[sl] = v
        return _
    lax.fori_loop(0, ROWS_PT // 16, acc_add, None)

    pltpu.sync_copy(accv, out_ref.at[c, pl.ds(s * ROWS_PT, ROWS_PT)])


_deg_kernel = functools.partial(
    pl.kernel,
    out_type=jax.ShapeDtypeStruct((NC, NPAD), jnp.float32),
    mesh=_mesh,
    scratch_types=[
        pltpu.VMEM((NPAD,), jnp.float32),
        pltpu.VMEM((2, 800), jnp.int32),
        pltpu.VMEM((NS, ROWS_PT), jnp.float32),
        pltpu.VMEM((ROWS_PT,), jnp.float32),
        pltpu.VMEM_SHARED((NS, NPAD), jnp.float32),
        pltpu.SemaphoreType.DMA,
        pltpu.SemaphoreType.DMA,
    ],
    compiler_params=_sc_params,
)(_deg_body)


G = 25
NBUF = 3


def _agg_body(srcs_ref, ge_ref, se_ref, out_ref, acc, bufg, bufs, rows, zbuf,
              semg, sems):
    c = lax.axis_index("c")
    s = lax.axis_index("s")
    n_chunks = EPT // K
    base = (c * NS + s) * n_chunks

    _zero_vmem_2d(zbuf, 16, 128)
    for k in range(ROWS_PT // 16):
        pltpu.sync_copy(zbuf, acc.at[pl.ds(s * ROWS_PT + k * 16, 16)])
    plsc.subcore_barrier()

    def rbuf(t):
        return rows.at[lax.rem(t, NBUF)]

    def gstart(t, dst):
        pltpu.async_copy(srcs_ref.at[bufg.at[t]], dst, semg)

    def gwait(t, dst):
        pltpu.make_async_copy(srcs_ref.at[bufg.at[t]], dst, semg).wait()

    def sstart(t):
        pltpu.async_copy(rbuf(t), acc.at[bufs.at[t]], sems, add=True)

    def swait(t):
        pltpu.make_async_copy(rbuf(t), acc.at[bufs.at[t]], sems).wait()

    def rnd(r, _):
        pltpu.sync_copy(ge_ref.at[pl.ds(base + r * G, G)], bufg)
        pltpu.sync_copy(se_ref.at[pl.ds(base + r * G, G)], bufs)
        gstart(0, rbuf(0))
        gstart(1, rbuf(1))

        def chunk(j, __):
            gwait(j, rbuf(j))
            sstart(j)

            @pl.when(j >= 1)
            def _():
                swait(j - 1)

            @pl.when(j <= G - 3)
            def _():
                gstart(j + 2, rbuf(j + 2))
            return __
        lax.fori_loop(0, G, chunk, None)
        swait(G - 1)
        return _
    lax.fori_loop(0, n_chunks // G, rnd, None)

    plsc.subcore_barrier()
    for k in range(ROWS_PT // 128):
        sl = pl.ds(s * ROWS_PT + k * 128, 128)
        pltpu.sync_copy(acc.at[sl], out_ref.at[c].at[sl])


_agg_kernel = functools.partial(
    pl.kernel,
    out_type=jax.ShapeDtypeStruct((NC, NPAD, D_IN), jnp.float32),
    mesh=_mesh,
    scratch_types=[
        pltpu.VMEM_SHARED((NPAD, D_IN), jnp.float32),
        pltpu.VMEM((G, K), jnp.int32),
        pltpu.VMEM((G, K), jnp.int32),
        pltpu.VMEM((NBUF, K, D_IN), jnp.float32),
        pltpu.VMEM((16, 128), jnp.float32),
        pltpu.SemaphoreType.DMA,
        pltpu.SemaphoreType.DMA,
    ],
    compiler_params=_sc_params,
)(_agg_body)


def _inv(d):
    return jnp.where(d > 0.0, lax.rsqrt(d), 0.0)


def _pre_body(x_ref, od_ref, id_ref, out_ref):
    x = x_ref[...]
    out_ref[0] = _inv(id_ref[...]) * x
    out_ref[1] = _inv(od_ref[...]) * x


def _mid_body(agg_ref, od_ref, id_ref, w1s_ref, b1s_ref, w1d_ref, b1d_ref,
              w2s_ref, w2d_ref, out_ref):
    oi = _inv(od_ref[...])
    ii = _inv(id_ref[...])
    a0 = oi * agg_ref[0]
    a1 = ii * agg_ref[1]
    h = ALPHA * (jnp.dot(a0, w1s_ref[...], preferred_element_type=jnp.float32)
                 + b1s_ref[...])
    h += (1.0 - ALPHA) * (jnp.dot(a1, w1d_ref[...],
                                  preferred_element_type=jnp.float32)
                          + b1d_ref[...])
    h = jnp.maximum(h, 0.0)
    out_ref[0] = ii * jnp.dot(h, w2s_ref[...],
                              preferred_element_type=jnp.float32)
    out_ref[1] = oi * jnp.dot(h, w2d_ref[...],
                              preferred_element_type=jnp.float32)


def _fin_body(agg_ref, od_ref, id_ref, b2s_ref, b2d_ref, out_ref):
    oi = _inv(od_ref[...])
    ii = _inv(id_ref[...])
    out_ref[...] = (ALPHA * (oi * agg_ref[0] + b2s_ref[...])
                    + (1.0 - ALPHA) * (ii * agg_ref[1] + b2d_ref[...]))


_BN = 1000


def _col(i):
    return (i, 0)


def _col3(i):
    return (0, i, 0)


def _rep2(i):
    return (0, 0)


@jax.jit
def kernel(x, edge_index, W1s, b1s, W1d, b1d, W2s, b2s, W2d, b2d):
    row = edge_index[0].astype(jnp.int32)
    col = edge_index[1].astype(jnp.int32)

    ge = jnp.concatenate([col, row + N]).reshape(2 * E // K, K)
    se = jnp.concatenate([row, col]).reshape(2 * E // K, K)
    de = jnp.concatenate([row, col])

    degs = _deg_kernel(de)
    od = degs[0, :N].reshape(N, 1)
    idg = degs[1, :N].reshape(N, 1)

    grid = N // _BN
    deg_spec = pl.BlockSpec((_BN, 1), _col)
    agg_spec = pl.BlockSpec((NC, _BN, D_IN), _col3)

    srcs1 = pl.pallas_call(
        _pre_body,
        grid=(grid,),
        in_specs=[pl.BlockSpec((_BN, D_IN), _col), deg_spec, deg_spec],
        out_specs=pl.BlockSpec((NC, _BN, D_IN), _col3),
        out_shape=jax.ShapeDtypeStruct((NC, N, D_IN), jnp.float32),
    )(x, od, idg)

    agg1 = _agg_kernel(srcs1.reshape(NC * N, D_IN), ge, se)

    srcs2 = pl.pallas_call(
        _mid_body,
        grid=(grid,),
        in_specs=[
            agg_spec, deg_spec, deg_spec,
            pl.BlockSpec((D_IN, D_HID), _rep2),
            pl.BlockSpec((1, D_HID), _rep2),
            pl.BlockSpec((D_IN, D_HID), _rep2),
            pl.BlockSpec((1, D_HID), _rep2),
            pl.BlockSpec((D_HID, D_OUT), _rep2),
            pl.BlockSpec((D_HID, D_OUT), _rep2),
        ],
        out_specs=pl.BlockSpec((NC, _BN, D_OUT), _col3),
        out_shape=jax.ShapeDtypeStruct((NC, N, D_OUT), jnp.float32),
    )(agg1, od, idg, W1s.T, b1s.reshape(1, -1), W1d.T, b1d.reshape(1, -1),
      W2s.T, W2d.T)

    agg2 = _agg_kernel(srcs2.reshape(NC * N, D_OUT), ge, se)

    out = pl.pallas_call(
        _fin_body,
        grid=(grid,),
        in_specs=[
            agg_spec, deg_spec, deg_spec,
            pl.BlockSpec((1, D_OUT), _rep2),
            pl.BlockSpec((1, D_OUT), _rep2),
        ],
        out_specs=pl.BlockSpec((_BN, D_OUT), _col),
        out_shape=jax.ShapeDtypeStruct((N, D_OUT), jnp.float32),
    )(agg2, od, idg, b2s.reshape(1, -1), b2d.reshape(1, -1))

    return out


# speedup vs baseline: 27.1374x; 1.0052x over previous
import functools

import jax
import jax.numpy as jnp
from jax import lax
from jax.experimental import pallas as pl
from jax.experimental.pallas import tpu as pltpu
from jax.experimental.pallas import tpu_sc as plsc

N = 10000
E = 320000
D_IN = 128
D_HID = 256
D_OUT = 128
ALPHA = 0.5

NPAD = 10240
NC = 2
NS = 16
K = 80
EPT = E // NS
ROWS_PT = NPAD // NS

_mesh = plsc.VectorSubcoreMesh(core_axis_name="c", subcore_axis_name="s")
_sc_params = pltpu.CompilerParams(needs_layout_passes=False,
                                  use_tc_tiling_on_sc=False)


def _zero_vmem_2d(ref, nrows, ncols):
    z16 = jnp.zeros((16,), jnp.float32)

    def body(i, _):
        def inner(j, __):
            ref[i, pl.ds(j * 16, 16)] = z16
            return __
        return lax.fori_loop(0, ncols // 16, inner, _)

    lax.fori_loop(0, nrows, body, None)


def _deg_body(de_ref, out_ref, hist, buf, tmp, accv, spm, semi, semt):
    c = lax.axis_index("c")
    s = lax.axis_index("s")
    z16 = jnp.zeros((16,), jnp.float32)
    ones16 = jnp.ones((16,), jnp.float32)

    def zh(i, _):
        hist[pl.ds(i * 16, 16)] = z16
        return _
    lax.fori_loop(0, NPAD // 16, zh, None)

    base = c * E + s * EPT
    n_stages = EPT // 800

    def istart(r):
        pltpu.async_copy(de_ref.at[pl.ds(base + r * 800, 800)],
                         buf.at[lax.rem(r, 2)], semi)

    def iwait(r):
        pltpu.make_async_copy(de_ref.at[pl.ds(base + r * 800, 800)],
                              buf.at[lax.rem(r, 2)], semi).wait()

    istart(0)

    def stage(r, _):
        iwait(r)

        @pl.when(r < n_stages - 1)
        def _():
            istart(r + 1)

        bslot = buf.at[lax.rem(r, 2)]

        def upd(j, __):
            idx = bslot[pl.ds(j * 16, 16)]
            plsc.addupdate_scatter(hist, [idx], ones16)
            return __
        return lax.fori_loop(0, 50, upd, _)
    lax.fori_loop(0, n_stages, stage, None)

    pltpu.sync_copy(hist, spm.at[s])
    plsc.subcore_barrier()

    for p in range(NS):
        pltpu.async_copy(spm.at[p, pl.ds(s * ROWS_PT, ROWS_PT)], tmp.at[p],
                         semt)
    for p in range(NS):
        pltpu.make_async_copy(spm.at[p, pl.ds(s * ROWS_PT, ROWS_PT)],
                              tmp.at[p], semt).wait()

    def acc_add(k, _):
        sl = pl.ds(k * 16, 16)
        v = tmp[0, sl]
        for p in range(1, NS):
            v = v + tmp[p, sl]
        accv[sl] = v
        return _
    lax.fori_loop(0, ROWS_PT // 16, acc_add, None)

    pltpu.sync_copy(accv, out_ref.at[c, pl.ds(s * ROWS_PT, ROWS_PT)])


_deg_kernel = functools.partial(
    pl.kernel,
    out_type=jax.ShapeDtypeStruct((NC, NPAD), jnp.float32),
    mesh=_mesh,
    scratch_types=[
        pltpu.VMEM((NPAD,), jnp.float32),
        pltpu.VMEM((2, 800), jnp.int32),
        pltpu.VMEM((NS, ROWS_PT), jnp.float32),
        pltpu.VMEM((ROWS_PT,), jnp.float32),
        pltpu.VMEM_SHARED((NS, NPAD), jnp.float32),
        pltpu.SemaphoreType.DMA,
        pltpu.SemaphoreType.DMA,
    ],
    compiler_params=_sc_params,
)(_deg_body)


G = 25
NBUF = 4


def _agg_body(srcs_ref, ge_ref, se_ref, out_ref, acc, bufg, bufs, rows, zbuf,
              semg, sems):
    c = lax.axis_index("c")
    s = lax.axis_index("s")
    n_chunks = EPT // K
    base = (c * NS + s) * n_chunks

    _zero_vmem_2d(zbuf, 16, 128)
    for k in range(ROWS_PT // 16):
        pltpu.sync_copy(zbuf, acc.at[pl.ds(s * ROWS_PT + k * 16, 16)])
    plsc.subcore_barrier()

    def rbuf(t):
        return rows.at[lax.rem(t, NBUF)]

    def gstart(t, dst):
        pltpu.async_copy(srcs_ref.at[bufg.at[t]], dst, semg)

    def gwait(t, dst):
        pltpu.make_async_copy(srcs_ref.at[bufg.at[t]], dst, semg).wait()

    def sstart(t):
        pltpu.async_copy(rbuf(t), acc.at[bufs.at[t]], sems, add=True)

    def swait(t):
        pltpu.make_async_copy(rbuf(t), acc.at[bufs.at[t]], sems).wait()

    def rnd(r, _):
        pltpu.sync_copy(ge_ref.at[pl.ds(base + r * G, G)], bufg)
        pltpu.sync_copy(se_ref.at[pl.ds(base + r * G, G)], bufs)
        gstart(0, rbuf(0))
        gstart(1, rbuf(1))
        gstart(2, rbuf(2))

        def chunk(j, __):
            gwait(j, rbuf(j))
            sstart(j)

            @pl.when(j >= 1)
            def _():
                swait(j - 1)

            @pl.when(j <= G - 4)
            def _():
                gstart(j + 3, rbuf(j + 3))
            return __
        lax.fori_loop(0, G, chunk, None)
        swait(G - 1)
        return _
    lax.fori_loop(0, n_chunks // G, rnd, None)

    plsc.subcore_barrier()
    for k in range(ROWS_PT // 128):
        sl = pl.ds(s * ROWS_PT + k * 128, 128)
        pltpu.sync_copy(acc.at[sl], out_ref.at[c].at[sl])


_agg_kernel = functools.partial(
    pl.kernel,
    out_type=jax.ShapeDtypeStruct((NC, NPAD, D_IN), jnp.float32),
    mesh=_mesh,
    scratch_types=[
        pltpu.VMEM_SHARED((NPAD, D_IN), jnp.float32),
        pltpu.VMEM((G, K), jnp.int32),
        pltpu.VMEM((G, K), jnp.int32),
        pltpu.VMEM((NBUF, K, D_IN), jnp.float32),
        pltpu.VMEM((16, 128), jnp.float32),
        pltpu.SemaphoreType.DMA,
        pltpu.SemaphoreType.DMA,
    ],
    compiler_params=_sc_params,
)(_agg_body)


def _inv(d):
    return jnp.where(d > 0.0, lax.rsqrt(d), 0.0)


def _pre_body(x_ref, od_ref, id_ref, out_ref):
    x = x_ref[...]
    out_ref[0] = _inv(id_ref[...]) * x
    out_ref[1] = _inv(od_ref[...]) * x


def _mid_body(agg_ref, od_ref, id_ref, w1s_ref, b1s_ref, w1d_ref, b1d_ref,
              w2s_ref, w2d_ref, out_ref):
    oi = _inv(od_ref[...])
    ii = _inv(id_ref[...])
    a0 = oi * agg_ref[0]
    a1 = ii * agg_ref[1]
    h = ALPHA * (jnp.dot(a0, w1s_ref[...], preferred_element_type=jnp.float32)
                 + b1s_ref[...])
    h += (1.0 - ALPHA) * (jnp.dot(a1, w1d_ref[...],
                                  preferred_element_type=jnp.float32)
                          + b1d_ref[...])
    h = jnp.maximum(h, 0.0)
    out_ref[0] = ii * jnp.dot(h, w2s_ref[...],
                              preferred_element_type=jnp.float32)
    out_ref[1] = oi * jnp.dot(h, w2d_ref[...],
                              preferred_element_type=jnp.float32)


def _fin_body(agg_ref, od_ref, id_ref, b2s_ref, b2d_ref, out_ref):
    oi = _inv(od_ref[...])
    ii = _inv(id_ref[...])
    out_ref[...] = (ALPHA * (oi * agg_ref[0] + b2s_ref[...])
                    + (1.0 - ALPHA) * (ii * agg_ref[1] + b2d_ref[...]))


_BN = 1000


def _col(i):
    return (i, 0)


def _col3(i):
    return (0, i, 0)


def _rep2(i):
    return (0, 0)


@jax.jit
def kernel(x, edge_index, W1s, b1s, W1d, b1d, W2s, b2s, W2d, b2d):
    row = edge_index[0].astype(jnp.int32)
    col = edge_index[1].astype(jnp.int32)

    ge = jnp.concatenate([col, row + N]).reshape(2 * E // K, K)
    se = jnp.concatenate([row, col]).reshape(2 * E // K, K)
    de = jnp.concatenate([row, col])

    degs = _deg_kernel(de)
    od = degs[0, :N].reshape(N, 1)
    idg = degs[1, :N].reshape(N, 1)

    grid = N // _BN
    deg_spec = pl.BlockSpec((_BN, 1), _col)
    agg_spec = pl.BlockSpec((NC, _BN, D_IN), _col3)

    srcs1 = pl.pallas_call(
        _pre_body,
        grid=(grid,),
        in_specs=[pl.BlockSpec((_BN, D_IN), _col), deg_spec, deg_spec],
        out_specs=pl.BlockSpec((NC, _BN, D_IN), _col3),
        out_shape=jax.ShapeDtypeStruct((NC, N, D_IN), jnp.float32),
    )(x, od, idg)

    agg1 = _agg_kernel(srcs1.reshape(NC * N, D_IN), ge, se)

    srcs2 = pl.pallas_call(
        _mid_body,
        grid=(grid,),
        in_specs=[
            agg_spec, deg_spec, deg_spec,
            pl.BlockSpec((D_IN, D_HID), _rep2),
            pl.BlockSpec((1, D_HID), _rep2),
            pl.BlockSpec((D_IN, D_HID), _rep2),
            pl.BlockSpec((1, D_HID), _rep2),
            pl.BlockSpec((D_HID, D_OUT), _rep2),
            pl.BlockSpec((D_HID, D_OUT), _rep2),
        ],
        out_specs=pl.BlockSpec((NC, _BN, D_OUT), _col3),
        out_shape=jax.ShapeDtypeStruct((NC, N, D_OUT), jnp.float32),
    )(agg1, od, idg, W1s.T, b1s.reshape(1, -1), W1d.T, b1d.reshape(1, -1),
      W2s.T, W2d.T)

    agg2 = _agg_kernel(srcs2.reshape(NC * N, D_OUT), ge, se)

    out = pl.pallas_call(
        _fin_body,
        grid=(grid,),
        in_specs=[
            agg_spec, deg_spec, deg_spec,
            pl.BlockSpec((1, D_OUT), _rep2),
            pl.BlockSpec((1, D_OUT), _rep2),
        ],
        out_specs=pl.BlockSpec((_BN, D_OUT), _col),
        out_shape=jax.ShapeDtypeStruct((N, D_OUT), jnp.float32),
    )(agg2, od, idg, b2s.reshape(1, -1), b2d.reshape(1, -1))

    return out
